# baseline (speedup 1.0000x reference)
import functools

import jax
import jax.numpy as jnp
from jax import lax
from jax.experimental import pallas as pl
from jax.experimental.pallas import tpu as pltpu
from jax.experimental.pallas import tpu_sc as plsc

N_USER = 25000
N_ITEM = 25000
N = N_USER + N_ITEM
NE = 800000
H = 16
NG = 4
NT = 16
NPAD = 51200
RPT = NPAD // NT
PT = 50176
EPAD = NT * PT
SB = 1024
ECH = 128
NCH = SB // ECH
NSB = PT // SB
CH = 128
INV3 = 1.0 / 3.0


def _body(x_in, src2, dst2, w2, outf, xa, xb,
          xs, acc, srcb, dstb, wb, gidx, rows, tmp, zbuf,
          esem, gsem, hsem, ssem):
    c = lax.axis_index("c")
    s = lax.axis_index("s")
    r0 = s * RPT
    e0 = pl.multiple_of((s * PT) // ECH, 8)

    def zb(i, _):
        z = jnp.zeros((16,), jnp.float32)
        zbuf[i, pl.ds(0, 16)] = z
        return 0
    lax.fori_loop(0, CH, zb, 0)

    for p in range(2):
        gbase = (2 * c + p) * NPAD

        pltpu.sync_copy(x_in.at[pl.ds(gbase + r0, RPT)],
                        xs.at[pl.ds(r0, RPT)])
        for t0 in range(0, RPT // CH, 8):
            zd = [pltpu.async_copy(
                      zbuf, acc.at[pl.ds(r0 + t * CH, CH)], ssem)
                  for t in range(t0, min(t0 + 8, RPT // CH))]
            for d in zd:
                d.wait()
        plsc.subcore_barrier()

        for layer in range(3):
            x_hbm = (x_in, xa, xb)[layer]
            pltpu.async_copy(src2.at[pl.ds(e0, NCH)], srcb.at[0], esem)
            pltpu.async_copy(dst2.at[pl.ds(e0, NCH)], dstb.at[0], esem)
            pltpu.async_copy(w2.at[pl.ds(e0, NCH)], wb.at[0], esem)

            def superchunk(sc_i, _):
                bi = sc_i % 2
                er = pl.multiple_of(e0 + sc_i * NCH, NCH)
                pltpu.make_async_copy(
                    src2.at[pl.ds(er, NCH)], srcb.at[bi], esem).wait()
                pltpu.make_async_copy(
                    dst2.at[pl.ds(er, NCH)], dstb.at[bi], esem).wait()
                pltpu.make_async_copy(
                    w2.at[pl.ds(er, NCH)], wb.at[bi], esem).wait()

                @pl.when(sc_i + 1 < NSB)
                def _():
                    nb = 1 - bi
                    er2 = pl.multiple_of(e0 + (sc_i + 1) * NCH, NCH)
                    pltpu.async_copy(
                        src2.at[pl.ds(er2, NCH)], srcb.at[nb], esem)
                    pltpu.async_copy(
                        dst2.at[pl.ds(er2, NCH)], dstb.at[nb], esem)
                    pltpu.async_copy(
                        w2.at[pl.ds(er2, NCH)], wb.at[nb], esem)

                def hix(i, _):
                    k = 1 + 2 * (i // (ECH // 16))
                    j = (i % (ECH // 16)) * 16
                    gidx[k, pl.ds(j, 16)] = srcb[bi, k, pl.ds(j, 16)] + gbase
                    return 0
                lax.fori_loop(0, (NCH // 2) * (ECH // 16), hix, 0)

                def issue(k):
                    if k % 2 == 0:
                        return pltpu.async_copy(
                            xs.at[srcb.at[bi, k]], rows.at[k % 4], gsem)
                    return pltpu.async_copy(
                        x_hbm.at[gidx.at[k]], rows.at[k % 4], hsem)

                gd = [None] * NCH
                sd = [None] * NCH
                gd[0] = issue(0)
                gd[1] = issue(1)
                for k in range(NCH):
                    if k >= 2:
                        sd[k - 2].wait()
                    if k + 2 < NCH:
                        gd[k + 2] = issue(k + 2)
                    gd[k].wait()
                    rb = rows.at[k % 4]

                    def scale(j, _):
                        wv = wb[bi, k, pl.ds(j * 16, 16)]
                        for l in range(16):
                            b = j * 16 + l
                            rb[b, pl.ds(0, 16)] = rb[b, pl.ds(0, 16)] * wv[l]
                        return 0
                    lax.fori_loop(0, ECH // 16, scale, 0, unroll=2)
                    sd[k] = pltpu.async_copy(
                        rb, acc.at[dstb.at[bi, k]], ssem, add=True)
                sd[NCH - 2].wait()
                sd[NCH - 1].wait()
                return 0
            lax.fori_loop(0, NSB, superchunk, 0)
            plsc.subcore_barrier()

            if layer < 2:
                x_dst = (xa, xb)[layer]
                pltpu.sync_copy(acc.at[pl.ds(r0, RPT)],
                                x_dst.at[pl.ds(gbase + r0, RPT)])
                pltpu.sync_copy(acc.at[pl.ds(r0, RPT)],
                                xs.at[pl.ds(r0, RPT)])
                for t0 in range(0, RPT // CH, 8):
                    zd = [pltpu.async_copy(
                              zbuf, acc.at[pl.ds(r0 + t * CH, CH)], ssem)
                          for t in range(t0, min(t0 + 8, RPT // CH))]
                    for d in zd:
                        d.wait()
            else:
                def drain3(t, _):
                    rr = r0 + t * CH
                    t1 = rows.at[2, pl.ds(0, CH)]
                    t2 = rows.at[3, pl.ds(0, CH)]
                    pltpu.sync_copy(acc.at[pl.ds(rr, CH)], tmp)
                    pltpu.sync_copy(xa.at[pl.ds(gbase + rr, CH)], t1)
                    pltpu.sync_copy(xb.at[pl.ds(gbase + rr, CH)], t2)

                    def mean3(b, _):
                        tmp[b, pl.ds(0, 16)] = (
                            t1[b, pl.ds(0, 16)] + t2[b, pl.ds(0, 16)]
                            + tmp[b, pl.ds(0, 16)]) * INV3
                        return 0
                    lax.fori_loop(0, CH, mean3, 0, unroll=4)
                    pltpu.sync_copy(tmp, outf.at[pl.ds(gbase + rr, CH)])
                    return 0
                lax.fori_loop(0, RPT // CH, drain3, 0)
            plsc.subcore_barrier()


_spmm3 = functools.partial(
    pl.kernel,
    out_type=(
        jax.ShapeDtypeStruct((NG * NPAD, H), jnp.float32),
        jax.ShapeDtypeStruct((NG * NPAD, H), jnp.float32),
        jax.ShapeDtypeStruct((NG * NPAD, H), jnp.float32),
    ),
    mesh=plsc.VectorSubcoreMesh(core_axis_name="c", subcore_axis_name="s",
                                num_cores=2, num_subcores=NT),
    compiler_params=pltpu.CompilerParams(use_tc_tiling_on_sc=False),
    scratch_types=(
        pltpu.VMEM_SHARED((NPAD, H), jnp.float32),
        pltpu.VMEM_SHARED((NPAD, H), jnp.float32),
        pltpu.VMEM((2, NCH, ECH), jnp.int32),
        pltpu.VMEM((2, NCH, ECH), jnp.int32),
        pltpu.VMEM((2, NCH, ECH), jnp.float32),
        pltpu.VMEM((NCH, ECH), jnp.int32),
        pltpu.VMEM((4, ECH, H), jnp.float32),
        pltpu.VMEM((CH, H), jnp.float32),
        pltpu.VMEM((CH, H), jnp.float32),
        pltpu.SemaphoreType.DMA,
        pltpu.SemaphoreType.DMA,
        pltpu.SemaphoreType.DMA,
        pltpu.SemaphoreType.DMA,
    ),
)(_body)


def kernel(user_emb, item_emb, edge_src, edge_dst, edge_w):
    ego = jnp.concatenate([user_emb, item_emb], axis=0)
    parts = [jnp.pad(ego[:, g * H:(g + 1) * H], ((0, NPAD - N), (0, 0)))
             for g in range(NG)]
    x_in = jnp.concatenate(parts, axis=0)
    src = jnp.pad(edge_src.astype(jnp.int32), (0, EPAD - NE))
    dst = jnp.pad(edge_dst.astype(jnp.int32), (0, EPAD - NE))
    w = jnp.pad(edge_w.astype(jnp.float32), (0, EPAD - NE))
    outf, _, _ = _spmm3(x_in,
                        src.reshape(EPAD // ECH, ECH),
                        dst.reshape(EPAD // ECH, ECH),
                        w.reshape(EPAD // ECH, ECH))
    final = jnp.concatenate(
        [outf[g * NPAD:g * NPAD + N] for g in range(NG)], axis=1)
    return final[:N_USER], final[N_USER:]


# speedup vs baseline: 1.1496x; 1.1496x over previous
import functools

import jax
import jax.numpy as jnp
from jax import lax
from jax.experimental import pallas as pl
from jax.experimental.pallas import tpu as pltpu
from jax.experimental.pallas import tpu_sc as plsc

N_USER = 25000
N_ITEM = 25000
N = N_USER + N_ITEM
NE = 800000
H = 16
NG = 4
NT = 16
NPAD = 51200
RPT = NPAD // NT
PT = 50176
EPAD = NT * PT
SB = 1024
ECH = 128
NCH = SB // ECH
NSB = PT // SB
CH = 128
INV3 = 1.0 / 3.0


def _body(x_in, src2, dst2, w2, outf, xa, xb,
          xs, acc, srcb, dstb, wb, rows, tmp, zbuf, esem, gsem, ssem):
    c = lax.axis_index("c")
    s = lax.axis_index("s")
    r0 = s * RPT
    e0 = pl.multiple_of((s * PT) // ECH, 8)

    def zb(i, _):
        z = jnp.zeros((16,), jnp.float32)
        zbuf[i, pl.ds(0, 16)] = z
        return 0
    lax.fori_loop(0, CH, zb, 0)

    for p in range(2):
        gbase = (2 * c + p) * NPAD

        pltpu.sync_copy(x_in.at[pl.ds(gbase + r0, RPT)],
                        xs.at[pl.ds(r0, RPT)])
        for t0 in range(0, RPT // CH, 8):
            zd = [pltpu.async_copy(
                      zbuf, acc.at[pl.ds(r0 + t * CH, CH)], ssem)
                  for t in range(t0, min(t0 + 8, RPT // CH))]
            for d in zd:
                d.wait()
        plsc.subcore_barrier()

        for layer in range(3):
            pltpu.async_copy(src2.at[pl.ds(e0, NCH)], srcb.at[0], esem)
            pltpu.async_copy(dst2.at[pl.ds(e0, NCH)], dstb.at[0], esem)
            pltpu.async_copy(w2.at[pl.ds(e0, NCH)], wb.at[0], esem)

            def superchunk(sc_i, _):
                bi = sc_i % 2
                er = pl.multiple_of(e0 + sc_i * NCH, NCH)
                pltpu.make_async_copy(
                    src2.at[pl.ds(er, NCH)], srcb.at[bi], esem).wait()
                pltpu.make_async_copy(
                    dst2.at[pl.ds(er, NCH)], dstb.at[bi], esem).wait()
                pltpu.make_async_copy(
                    w2.at[pl.ds(er, NCH)], wb.at[bi], esem).wait()

                @pl.when(sc_i + 1 < NSB)
                def _():
                    nb = 1 - bi
                    er2 = pl.multiple_of(e0 + (sc_i + 1) * NCH, NCH)
                    pltpu.async_copy(
                        src2.at[pl.ds(er2, NCH)], srcb.at[nb], esem)
                    pltpu.async_copy(
                        dst2.at[pl.ds(er2, NCH)], dstb.at[nb], esem)
                    pltpu.async_copy(
                        w2.at[pl.ds(er2, NCH)], wb.at[nb], esem)

                def gix(k):
                    return srcb.at[bi, k]

                gd = [None] * NCH
                sd = [None] * NCH
                gd[0] = pltpu.async_copy(xs.at[gix(0)], rows.at[0], gsem)
                gd[1] = pltpu.async_copy(xs.at[gix(1)], rows.at[1], gsem)
                for k in range(NCH):
                    if k >= 2:
                        sd[k - 2].wait()
                    if k + 2 < NCH:
                        gd[k + 2] = pltpu.async_copy(
                            xs.at[gix(k + 2)], rows.at[(k + 2) % 4], gsem)
                    gd[k].wait()
                    rb = rows.at[k % 4]

                    def scale(j, _):
                        wv = wb[bi, k, pl.ds(j * 16, 16)]
                        for l in range(16):
                            b = j * 16 + l
                            rb[b, pl.ds(0, 16)] = rb[b, pl.ds(0, 16)] * wv[l]
                        return 0
                    lax.fori_loop(0, ECH // 16, scale, 0, unroll=2)
                    sd[k] = pltpu.async_copy(
                        rb, acc.at[dstb.at[bi, k]], ssem, add=True)
                sd[NCH - 2].wait()
                sd[NCH - 1].wait()
                return 0
            lax.fori_loop(0, NSB, superchunk, 0)
            plsc.subcore_barrier()

            if layer < 2:
                x_dst = (xa, xb)[layer]
                pltpu.sync_copy(acc.at[pl.ds(r0, RPT)],
                                x_dst.at[pl.ds(gbase + r0, RPT)])
                pltpu.sync_copy(acc.at[pl.ds(r0, RPT)],
                                xs.at[pl.ds(r0, RPT)])
                for t0 in range(0, RPT // CH, 8):
                    zd = [pltpu.async_copy(
                              zbuf, acc.at[pl.ds(r0 + t * CH, CH)], ssem)
                          for t in range(t0, min(t0 + 8, RPT // CH))]
                    for d in zd:
                        d.wait()
            else:
                def drain3(t, _):
                    rr = r0 + t * CH
                    t1 = rows.at[2, pl.ds(0, CH)]
                    t2 = rows.at[3, pl.ds(0, CH)]
                    pltpu.sync_copy(acc.at[pl.ds(rr, CH)], tmp)
                    pltpu.sync_copy(xa.at[pl.ds(gbase + rr, CH)], t1)
                    pltpu.sync_copy(xb.at[pl.ds(gbase + rr, CH)], t2)

                    def mean3(b, _):
                        tmp[b, pl.ds(0, 16)] = (
                            t1[b, pl.ds(0, 16)] + t2[b, pl.ds(0, 16)]
                            + tmp[b, pl.ds(0, 16)]) * INV3
                        return 0
                    lax.fori_loop(0, CH, mean3, 0, unroll=4)
                    pltpu.sync_copy(tmp, outf.at[pl.ds(gbase + rr, CH)])
                    return 0
                lax.fori_loop(0, RPT // CH, drain3, 0)
            plsc.subcore_barrier()


_spmm3 = functools.partial(
    pl.kernel,
    out_type=(
        jax.ShapeDtypeStruct((NG * NPAD, H), jnp.float32),
        jax.ShapeDtypeStruct((NG * NPAD, H), jnp.float32),
        jax.ShapeDtypeStruct((NG * NPAD, H), jnp.float32),
    ),
    mesh=plsc.VectorSubcoreMesh(core_axis_name="c", subcore_axis_name="s",
                                num_cores=2, num_subcores=NT),
    compiler_params=pltpu.CompilerParams(use_tc_tiling_on_sc=False),
    scratch_types=(
        pltpu.VMEM_SHARED((NPAD, H), jnp.float32),
        pltpu.VMEM_SHARED((NPAD, H), jnp.float32),
        pltpu.VMEM((2, NCH, ECH), jnp.int32),
        pltpu.VMEM((2, NCH, ECH), jnp.int32),
        pltpu.VMEM((2, NCH, ECH), jnp.float32),
        pltpu.VMEM((4, ECH, H), jnp.float32),
        pltpu.VMEM((CH, H), jnp.float32),
        pltpu.VMEM((CH, H), jnp.float32),
        pltpu.SemaphoreType.DMA,
        pltpu.SemaphoreType.DMA,
        pltpu.SemaphoreType.DMA,
    ),
)(_body)


def kernel(user_emb, item_emb, edge_src, edge_dst, edge_w):
    ego = jnp.concatenate([user_emb, item_emb], axis=0)
    parts = [jnp.pad(ego[:, g * H:(g + 1) * H], ((0, NPAD - N), (0, 0)))
             for g in range(NG)]
    x_in = jnp.concatenate(parts, axis=0)
    src = jnp.pad(edge_src.astype(jnp.int32), (0, EPAD - NE))
    dst = jnp.pad(edge_dst.astype(jnp.int32), (0, EPAD - NE))
    w = jnp.pad(edge_w.astype(jnp.float32), (0, EPAD - NE))
    outf, _, _ = _spmm3(x_in,
                        src.reshape(EPAD // ECH, ECH),
                        dst.reshape(EPAD // ECH, ECH),
                        w.reshape(EPAD // ECH, ECH))
    final = jnp.concatenate(
        [outf[g * NPAD:g * NPAD + N] for g in range(NG)], axis=1)
    return final[:N_USER], final[N_USER:]
